# baseline (speedup 1.0000x reference)
import functools

import jax
import jax.numpy as jnp
from jax import lax
from jax.experimental import pallas as pl
from jax.experimental.pallas import tpu as pltpu
from jax.experimental.pallas import tpu_sc as plsc

LANES = 16
CHUNK = 64
KBLK = 768 // LANES


def _lanesum(v):
    idx = lax.iota(jnp.int32, LANES)
    for sh in (1, 2, 4, 8):
        v = v + v.at[idx ^ sh].get(mode="promise_in_bounds")
    return v


@functools.cache
def _make_router(batch, embed):
    info = plsc.get_sparse_core_info()
    nc, ns = info.num_cores, info.num_subcores
    nw = nc * ns
    assert batch % (8 * nw) == 0 and embed == 768
    b_per_w = batch // nw
    n_chunks = b_per_w // CHUNK
    mesh = plsc.VectorSubcoreMesh(core_axis_name="c", subcore_axis_name="s")

    out_ty = jax.ShapeDtypeStruct((batch,), jnp.float32)

    @functools.partial(
        pl.kernel,
        mesh=mesh,
        compiler_params=pltpu.CompilerParams(use_tc_tiling_on_sc=False),
        out_type=(out_ty, out_ty),
        scratch_types=[
            pltpu.VMEM((b_per_w,), jnp.int32),
            pltpu.VMEM((CHUNK, embed), jnp.float32),
            pltpu.VMEM((CHUNK, embed), jnp.float32),
            pltpu.VMEM((2, embed), jnp.float32),
            pltpu.VMEM((2, LANES), jnp.float32),
            pltpu.VMEM((b_per_w,), jnp.float32),
            pltpu.VMEM((b_per_w,), jnp.float32),
            pltpu.SemaphoreType.DMA,
            pltpu.SemaphoreType.DMA,
        ],
    )
    def router(table_hbm, ids_hbm, w_hbm, b_hbm, out0_hbm, out1_hbm,
               idx_v, buf_a, buf_b, w_v, b_v, o0_v, o1_v, sem_a, sem_b):
        wid = lax.axis_index("s") * nc + lax.axis_index("c")
        base = wid * b_per_w
        pltpu.sync_copy(ids_hbm.at[pl.ds(base, b_per_w)], idx_v)
        pltpu.sync_copy(w_hbm, w_v)
        pltpu.sync_copy(b_hbm, b_v)

        bufs = (buf_a, buf_b)
        sems = (sem_a, sem_b)
        copies = [None] * n_chunks
        copies[0] = pltpu.async_copy(
            table_hbm.at[idx_v.at[pl.ds(0, CHUNK)]], bufs[0], sems[0])

        lane_ids = lax.iota(jnp.int32, LANES)
        b0 = b_v[0, :]
        b1 = b_v[1, :]

        for ch in range(n_chunks):
            buf = bufs[ch % 2]
            copies[ch].wait()
            if ch + 1 < n_chunks:
                copies[ch + 1] = pltpu.async_copy(
                    table_hbm.at[idx_v.at[pl.ds((ch + 1) * CHUNK, CHUNK)]],
                    bufs[(ch + 1) % 2], sems[(ch + 1) % 2])

            for g in range(CHUNK // LANES):
                def body(kv, accs):
                    a0, a1 = accs
                    w0 = w_v[0, pl.ds(kv * LANES, LANES)]
                    w1 = w_v[1, pl.ds(kv * LANES, LANES)]
                    new0 = []
                    new1 = []
                    for r in range(LANES):
                        ev = buf[g * LANES + r, pl.ds(kv * LANES, LANES)]
                        new0.append(a0[r] + ev * w0)
                        new1.append(a1[r] + ev * w1)
                    return tuple(new0), tuple(new1)

                acc0 = tuple(b0 for _ in range(LANES))
                acc1 = tuple(b1 for _ in range(LANES))
                acc0, acc1 = lax.fori_loop(0, KBLK, body, (acc0, acc1))
                row0 = ch * CHUNK + g * LANES
                res0 = jnp.zeros((LANES,), jnp.float32)
                res1 = jnp.zeros((LANES,), jnp.float32)
                for r in range(LANES):
                    m = lane_ids == r
                    res0 = jnp.where(m, _lanesum(acc0[r]), res0)
                    res1 = jnp.where(m, _lanesum(acc1[r]), res1)
                o0_v[pl.ds(row0, LANES)] = res0
                o1_v[pl.ds(row0, LANES)] = res1

        pltpu.sync_copy(o0_v, out0_hbm.at[pl.ds(base, b_per_w)])
        pltpu.sync_copy(o1_v, out1_hbm.at[pl.ds(base, b_per_w)])

    return router


def kernel(input_ids, token_emb, fc_w, fc_b):
    batch = input_ids.shape[0]
    embed = token_emb.shape[1]
    ids = input_ids[:, 0].astype(jnp.int32)
    w_t = fc_w.T
    b_lane0 = jnp.zeros((2, LANES), jnp.float32).at[:, 0].set(fc_b)
    out0, out1 = _make_router(batch, embed)(token_emb, ids, w_t, b_lane0)
    return jnp.stack([out0, out1], axis=1)


# speedup vs baseline: 3.3761x; 3.3761x over previous
import functools

import jax
import jax.numpy as jnp
from jax import lax
from jax.experimental import pallas as pl
from jax.experimental.pallas import tpu as pltpu
from jax.experimental.pallas import tpu_sc as plsc

LANES = 16
CHUNK = 64
KBLK = 768 // LANES


def _lanesum(v):
    idx = lax.iota(jnp.int32, LANES)
    for sh in (1, 2, 4, 8):
        v = v + v.at[idx ^ sh].get(mode="promise_in_bounds")
    return v


@functools.cache
def _make_router(batch, embed):
    info = plsc.get_sparse_core_info()
    nc, ns = info.num_cores, info.num_subcores
    nw = nc * ns
    assert batch % (8 * nw) == 0 and embed == 768
    b_per_w = batch // nw
    n_chunks = b_per_w // CHUNK
    mesh = plsc.VectorSubcoreMesh(core_axis_name="c", subcore_axis_name="s")

    out_ty = jax.ShapeDtypeStruct((batch,), jnp.float32)

    @functools.partial(
        pl.kernel,
        mesh=mesh,
        compiler_params=pltpu.CompilerParams(use_tc_tiling_on_sc=True),
        out_type=(out_ty, out_ty),
        scratch_types=[
            pltpu.VMEM((b_per_w,), jnp.int32),
            pltpu.VMEM((CHUNK, embed), jnp.float32),
            pltpu.VMEM((CHUNK, embed), jnp.float32),
            pltpu.VMEM((8, embed), jnp.float32),
            pltpu.VMEM((8, 128), jnp.float32),
            pltpu.VMEM((b_per_w,), jnp.float32),
            pltpu.VMEM((b_per_w,), jnp.float32),
            pltpu.SemaphoreType.DMA,
            pltpu.SemaphoreType.DMA,
        ],
    )
    def router(table_hbm, ids_hbm, w_hbm, b_hbm, out0_hbm, out1_hbm,
               idx_v, buf_a, buf_b, w_v, b_v, o0_v, o1_v, sem_a, sem_b):
        wid = lax.axis_index("s") * nc + lax.axis_index("c")
        base = wid * b_per_w
        pltpu.sync_copy(ids_hbm.at[pl.ds(base, b_per_w)], idx_v)
        pltpu.sync_copy(w_hbm, w_v)
        pltpu.sync_copy(b_hbm, b_v)

        bufs = (buf_a, buf_b)
        sems = (sem_a, sem_b)
        copies = [None] * n_chunks
        copies[0] = pltpu.async_copy(
            table_hbm.at[idx_v.at[pl.ds(0, CHUNK)]], bufs[0], sems[0])

        lane_ids = lax.iota(jnp.int32, LANES)
        b0 = b_v[0, pl.ds(0, LANES)]
        b1 = b_v[1, pl.ds(0, LANES)]

        for ch in range(n_chunks):
            buf = bufs[ch % 2]
            copies[ch].wait()
            if ch + 1 < n_chunks:
                copies[ch + 1] = pltpu.async_copy(
                    table_hbm.at[idx_v.at[pl.ds((ch + 1) * CHUNK, CHUNK)]],
                    bufs[(ch + 1) % 2], sems[(ch + 1) % 2])

            for g in range(CHUNK // LANES):
                def body(kv, accs):
                    a0, a1 = accs
                    w0 = w_v[0, pl.ds(kv * LANES, LANES)]
                    w1 = w_v[1, pl.ds(kv * LANES, LANES)]
                    new0 = []
                    new1 = []
                    for r in range(LANES):
                        ev = buf[g * LANES + r, pl.ds(kv * LANES, LANES)]
                        new0.append(a0[r] + ev * w0)
                        new1.append(a1[r] + ev * w1)
                    return tuple(new0), tuple(new1)

                acc0 = tuple(b0 for _ in range(LANES))
                acc1 = tuple(b1 for _ in range(LANES))
                acc0, acc1 = lax.fori_loop(0, KBLK, body, (acc0, acc1))
                row0 = ch * CHUNK + g * LANES
                res0 = jnp.zeros((LANES,), jnp.float32)
                res1 = jnp.zeros((LANES,), jnp.float32)
                for r in range(LANES):
                    m = lane_ids == r
                    res0 = jnp.where(m, _lanesum(acc0[r]), res0)
                    res1 = jnp.where(m, _lanesum(acc1[r]), res1)
                o0_v[pl.ds(row0, LANES)] = res0
                o1_v[pl.ds(row0, LANES)] = res1

        pltpu.sync_copy(o0_v, out0_hbm.at[pl.ds(base, b_per_w)])
        pltpu.sync_copy(o1_v, out1_hbm.at[pl.ds(base, b_per_w)])

    return router


def kernel(input_ids, token_emb, fc_w, fc_b):
    batch = input_ids.shape[0]
    embed = token_emb.shape[1]
    ids = input_ids[:, 0].astype(jnp.int32)
    w_t = jnp.zeros((8, embed), jnp.float32).at[:2, :].set(fc_w.T)
    b_lane0 = jnp.zeros((8, 128), jnp.float32).at[:2, 0].set(fc_b)
    out0, out1 = _make_router(batch, embed)(token_emb, ids, w_t, b_lane0)
    return jnp.stack([out0, out1], axis=1)
